# baseline (speedup 1.0000x reference)
import functools

import jax
import jax.numpy as jnp
from jax import lax
from jax.experimental import pallas as pl
from jax.experimental.pallas import tpu as pltpu
from jax.experimental.pallas import tpu_sc as plsc

B = 16384
VOCAB = 100000
EMB = 64
HID = 64
NT = 5

NC = 2
NS = 16
NW = NC * NS
BPW = B // NW
CHUNK = 128
NCHUNK = BPW // CHUNK
IDX_ROWS = 8

VC = 1408
NVB = 36
SPLIT = NVB * VC


def _tp_body(lo_ref, hi_ref, o_ref):
    o_ref[...] = jnp.concatenate([lo_ref[...].T, hi_ref[...].T], axis=-1)


def _transpose_pack(tab_t):
    return pl.pallas_call(
        _tp_body,
        grid=(NVB,),
        in_specs=[
            pl.BlockSpec((EMB, VC), lambda i: (0, i)),
            pl.BlockSpec((EMB, VC), lambda i: (0, i + NVB)),
        ],
        out_specs=pl.BlockSpec((VC, 2 * EMB), lambda i: (i, 0)),
        out_shape=jax.ShapeDtypeStruct((SPLIT, 2 * EMB), jnp.float32),
    )(tab_t, tab_t)


@functools.lru_cache(maxsize=None)
def _make_sc_gather():
    mesh = plsc.VectorSubcoreMesh(core_axis_name="c", subcore_axis_name="s")

    @functools.partial(
        pl.kernel,
        mesh=mesh,
        out_type=jax.ShapeDtypeStruct((B, 2 * EMB), jnp.float32),
        scratch_types=[
            pltpu.VMEM((IDX_ROWS, CHUNK), jnp.int32),
            pltpu.VMEM((NCHUNK, CHUNK, 2 * EMB), jnp.float32),
            pltpu.SemaphoreType.DMA((NCHUNK,)),
            pltpu.SemaphoreType.DMA((NCHUNK,)),
        ],
    )
    def sc_gather(idx_hbm, tab_hbm, out_hbm, idx_v, rows_v, gsem, wsem):
        wid = lax.axis_index("s") * NC + lax.axis_index("c")
        base = wid * BPW
        pltpu.sync_copy(idx_hbm.at[wid], idx_v)
        gh = [None] * NCHUNK
        wh = [None] * NCHUNK
        for c in range(NCHUNK):
            gh[c] = pltpu.async_copy(
                tab_hbm.at[idx_v.at[c]], rows_v.at[c], gsem.at[c])
        for c in range(NCHUNK):
            gh[c].wait()
            wh[c] = pltpu.async_copy(
                rows_v.at[c], out_hbm.at[pl.ds(base + c * CHUNK, CHUNK)],
                wsem.at[c])
        for c in range(NCHUNK):
            wh[c].wait()

    return sc_gather


def _mm_body(g0, g1, g2, g3, g4, bsc_ref, w_ref, b_ref, o_ref):
    bm = o_ref.shape[0]
    half = (lax.broadcasted_iota(jnp.int32, (bm, 2 * EMB), 1) >= EMB)
    half = half.astype(jnp.int32)
    acc = b_ref[...].astype(jnp.float32)
    for t, g_ref in enumerate((g0, g1, g2, g3, g4)):
        par = (bsc_ref[:, t:t + 1] >= SPLIT).astype(jnp.int32)
        mask = half == par
        gt = jnp.where(mask, g_ref[...], 0.0)
        acc = acc + jnp.dot(gt, w_ref[t], preferred_element_type=jnp.float32)
    o_ref[...] = acc


def _project(gs, bsc, w_p, b_r):
    BM = 2048
    return pl.pallas_call(
        _mm_body,
        grid=(B // BM,),
        in_specs=[pl.BlockSpec((BM, 2 * EMB), lambda i: (i, 0))] * NT + [
            pl.BlockSpec((BM, NT), lambda i: (i, 0)),
            pl.BlockSpec((NT, 2 * EMB, HID), lambda i: (0, 0, 0)),
            pl.BlockSpec((1, HID), lambda i: (0, 0)),
        ],
        out_specs=pl.BlockSpec((BM, HID), lambda i: (i, 0)),
        out_shape=jax.ShapeDtypeStruct((B, HID), jnp.float32),
    )(*gs, bsc, w_p, b_r)


def kernel(batch_seq_cat, pop_tab, leng_tab, area_tab, lon_tab, lat_tab, W, b):
    bsc = batch_seq_cat.astype(jnp.int32)
    ridx = jnp.where(bsc < SPLIT, bsc, bsc - SPLIT)
    pidx = ridx.T.reshape(NT, NW, NCHUNK, CHUNK)
    pidx = jnp.concatenate(
        [pidx, jnp.zeros((NT, NW, IDX_ROWS - NCHUNK, CHUNK), jnp.int32)],
        axis=2)
    gather = _make_sc_gather()
    gs = []
    for t, tab in enumerate((pop_tab, leng_tab, area_tab, lon_tab, lat_tab)):
        gs.append(gather(pidx[t], _transpose_pack(tab.T)))
    w_p = jnp.concatenate([W.reshape(NT, EMB, HID)] * 2, axis=1)
    return _project(gs, bsc, w_p, b.reshape(1, HID))


# speedup vs baseline: 1.1103x; 1.1103x over previous
import functools

import jax
import jax.numpy as jnp
from jax import lax
from jax.experimental import pallas as pl
from jax.experimental.pallas import tpu as pltpu
from jax.experimental.pallas import tpu_sc as plsc

B = 16384
VOCAB = 100000
EMB = 64
HID = 64
NT = 5

NC = 2
NS = 16
NW = NC * NS
BPW = B // NW
CHUNK = 128
NCHUNK = BPW // CHUNK
IDX_ROWS = 8

VC = 2048
NVB = 25
SPLIT = NVB * VC
LASTB = (VOCAB - 1) // VC


def _tp_body(lo_ref, hi_ref, o_ref):
    o_ref[...] = jnp.concatenate([lo_ref[...].T, hi_ref[...].T], axis=-1)


def _transpose_pack(tab_t):
    return pl.pallas_call(
        _tp_body,
        grid=(NVB,),
        in_specs=[
            pl.BlockSpec((EMB, VC), lambda i: (0, i)),
            pl.BlockSpec((EMB, VC),
                         lambda i: (0, jnp.minimum(i + NVB, LASTB))),
        ],
        out_specs=pl.BlockSpec((VC, 2 * EMB), lambda i: (i, 0)),
        out_shape=jax.ShapeDtypeStruct((SPLIT, 2 * EMB), jnp.float32),
    )(tab_t, tab_t)


@functools.lru_cache(maxsize=None)
def _make_sc_gather():
    mesh = plsc.VectorSubcoreMesh(core_axis_name="c", subcore_axis_name="s")

    @functools.partial(
        pl.kernel,
        mesh=mesh,
        out_type=jax.ShapeDtypeStruct((B, 2 * EMB), jnp.float32),
        scratch_types=[
            pltpu.VMEM((IDX_ROWS, CHUNK), jnp.int32),
            pltpu.VMEM((NCHUNK, CHUNK, 2 * EMB), jnp.float32),
            pltpu.SemaphoreType.DMA((NCHUNK,)),
            pltpu.SemaphoreType.DMA((NCHUNK,)),
        ],
    )
    def sc_gather(idx_hbm, tab_hbm, out_hbm, idx_v, rows_v, gsem, wsem):
        wid = lax.axis_index("s") * NC + lax.axis_index("c")
        base = wid * BPW
        pltpu.sync_copy(idx_hbm.at[wid], idx_v)
        gh = [None] * NCHUNK
        wh = [None] * NCHUNK
        for c in range(NCHUNK):
            gh[c] = pltpu.async_copy(
                tab_hbm.at[idx_v.at[c]], rows_v.at[c], gsem.at[c])
        for c in range(NCHUNK):
            gh[c].wait()
            wh[c] = pltpu.async_copy(
                rows_v.at[c], out_hbm.at[pl.ds(base + c * CHUNK, CHUNK)],
                wsem.at[c])
        for c in range(NCHUNK):
            wh[c].wait()

    return sc_gather


def _mm_body(g0, g1, g2, g3, g4, bsc_ref, w_ref, b_ref, o_ref):
    bm = o_ref.shape[0]
    half = (lax.broadcasted_iota(jnp.int32, (bm, 2 * EMB), 1) >= EMB)
    half = half.astype(jnp.int32)
    acc = b_ref[...].astype(jnp.float32)
    for t, g_ref in enumerate((g0, g1, g2, g3, g4)):
        par = (bsc_ref[:, t:t + 1] >= SPLIT).astype(jnp.int32)
        mask = half == par
        gt = jnp.where(mask, g_ref[...], 0.0)
        acc = acc + jnp.dot(gt, w_ref[t], preferred_element_type=jnp.float32)
    o_ref[...] = acc


def _project(gs, bsc, w_p, b_r):
    BM = 2048
    return pl.pallas_call(
        _mm_body,
        grid=(B // BM,),
        in_specs=[pl.BlockSpec((BM, 2 * EMB), lambda i: (i, 0))] * NT + [
            pl.BlockSpec((BM, NT), lambda i: (i, 0)),
            pl.BlockSpec((NT, 2 * EMB, HID), lambda i: (0, 0, 0)),
            pl.BlockSpec((1, HID), lambda i: (0, 0)),
        ],
        out_specs=pl.BlockSpec((BM, HID), lambda i: (i, 0)),
        out_shape=jax.ShapeDtypeStruct((B, HID), jnp.float32),
    )(*gs, bsc, w_p, b_r)


def kernel(batch_seq_cat, pop_tab, leng_tab, area_tab, lon_tab, lat_tab, W, b):
    bsc = batch_seq_cat.astype(jnp.int32)
    ridx = jnp.where(bsc < SPLIT, bsc, bsc - SPLIT)
    pidx = ridx.T.reshape(NT, NW, NCHUNK, CHUNK)
    pidx = jnp.concatenate(
        [pidx, jnp.zeros((NT, NW, IDX_ROWS - NCHUNK, CHUNK), jnp.int32)],
        axis=2)
    gather = _make_sc_gather()
    gs = []
    for t, tab in enumerate((pop_tab, leng_tab, area_tab, lon_tab, lat_tab)):
        gs.append(gather(pidx[t], _transpose_pack(tab.T)))
    w_p = jnp.concatenate([W.reshape(NT, EMB, HID)] * 2, axis=1)
    return _project(gs, bsc, w_p, b.reshape(1, HID))


# speedup vs baseline: 1.2058x; 1.0861x over previous
import functools

import jax
import jax.numpy as jnp
from jax import lax
from jax.experimental import pallas as pl
from jax.experimental.pallas import tpu as pltpu
from jax.experimental.pallas import tpu_sc as plsc

B = 16384
VOCAB = 100000
EMB = 64
HID = 64
NT = 5

NC = 2
NS = 16
NW = NC * NS
BPW = B // NW
CHUNK = 128
NCHUNK = BPW // CHUNK
IDX_ROWS = 8

VC = 3072
NVB = 17
SPLIT = NVB * VC
LASTB = (VOCAB - 1) // VC


def _tp_body(lo_ref, hi_ref, o_ref):
    o_ref[...] = jnp.concatenate([lo_ref[...].T, hi_ref[...].T], axis=-1)


def _transpose_pack(tab_t):
    return pl.pallas_call(
        _tp_body,
        grid=(NVB,),
        in_specs=[
            pl.BlockSpec((EMB, VC), lambda i: (0, i)),
            pl.BlockSpec((EMB, VC),
                         lambda i: (0, jnp.minimum(i + NVB, LASTB))),
        ],
        out_specs=pl.BlockSpec((VC, 2 * EMB), lambda i: (i, 0)),
        out_shape=jax.ShapeDtypeStruct((SPLIT, 2 * EMB), jnp.float32),
    )(tab_t, tab_t)


@functools.lru_cache(maxsize=None)
def _make_sc_gather():
    mesh = plsc.VectorSubcoreMesh(core_axis_name="c", subcore_axis_name="s")

    @functools.partial(
        pl.kernel,
        mesh=mesh,
        out_type=jax.ShapeDtypeStruct((B, 2 * EMB), jnp.float32),
        scratch_types=[
            pltpu.VMEM((IDX_ROWS, CHUNK), jnp.int32),
            pltpu.VMEM((NCHUNK, CHUNK, 2 * EMB), jnp.float32),
            pltpu.SemaphoreType.DMA((NCHUNK,)),
            pltpu.SemaphoreType.DMA((NCHUNK,)),
        ],
    )
    def sc_gather(idx_hbm, tab_hbm, out_hbm, idx_v, rows_v, gsem, wsem):
        wid = lax.axis_index("s") * NC + lax.axis_index("c")
        base = wid * BPW
        pltpu.sync_copy(idx_hbm.at[wid], idx_v)
        gh = [None] * NCHUNK
        wh = [None] * NCHUNK
        for c in range(NCHUNK):
            gh[c] = pltpu.async_copy(
                tab_hbm.at[idx_v.at[c]], rows_v.at[c], gsem.at[c])
        for c in range(NCHUNK):
            gh[c].wait()
            wh[c] = pltpu.async_copy(
                rows_v.at[c], out_hbm.at[pl.ds(base + c * CHUNK, CHUNK)],
                wsem.at[c])
        for c in range(NCHUNK):
            wh[c].wait()

    return sc_gather


def _mm_body(g0, g1, g2, g3, g4, bsc_ref, w_ref, b_ref, o_ref):
    bm = o_ref.shape[0]
    half = (lax.broadcasted_iota(jnp.int32, (bm, 2 * EMB), 1) >= EMB)
    half = half.astype(jnp.int32)
    acc = b_ref[...].astype(jnp.float32)
    for t, g_ref in enumerate((g0, g1, g2, g3, g4)):
        par = (bsc_ref[:, t:t + 1] >= SPLIT).astype(jnp.int32)
        mask = half == par
        gt = jnp.where(mask, g_ref[...], 0.0)
        acc = acc + jnp.dot(gt, w_ref[t], preferred_element_type=jnp.float32)
    o_ref[...] = acc


def _project(gs, bsc, w_p, b_r):
    BM = 2048
    return pl.pallas_call(
        _mm_body,
        grid=(B // BM,),
        in_specs=[pl.BlockSpec((BM, 2 * EMB), lambda i: (i, 0))] * NT + [
            pl.BlockSpec((BM, NT), lambda i: (i, 0)),
            pl.BlockSpec((NT, 2 * EMB, HID), lambda i: (0, 0, 0)),
            pl.BlockSpec((1, HID), lambda i: (0, 0)),
        ],
        out_specs=pl.BlockSpec((BM, HID), lambda i: (i, 0)),
        out_shape=jax.ShapeDtypeStruct((B, HID), jnp.float32),
    )(*gs, bsc, w_p, b_r)


def kernel(batch_seq_cat, pop_tab, leng_tab, area_tab, lon_tab, lat_tab, W, b):
    bsc = batch_seq_cat.astype(jnp.int32)
    ridx = jnp.where(bsc < SPLIT, bsc, bsc - SPLIT)
    pidx = ridx.T.reshape(NT, NW, NCHUNK, CHUNK)
    pidx = jnp.concatenate(
        [pidx, jnp.zeros((NT, NW, IDX_ROWS - NCHUNK, CHUNK), jnp.int32)],
        axis=2)
    gather = _make_sc_gather()
    gs = []
    for t, tab in enumerate((pop_tab, leng_tab, area_tab, lon_tab, lat_tab)):
        gs.append(gather(pidx[t], _transpose_pack(tab.T)))
    w_p = jnp.concatenate([W.reshape(NT, EMB, HID)] * 2, axis=1)
    return _project(gs, bsc, w_p, b.reshape(1, HID))


# speedup vs baseline: 1.2593x; 1.0443x over previous
import functools

import jax
import jax.numpy as jnp
from jax import lax
from jax.experimental import pallas as pl
from jax.experimental.pallas import tpu as pltpu
from jax.experimental.pallas import tpu_sc as plsc

B = 16384
VOCAB = 100000
EMB = 64
HID = 64
NT = 5

NC = 2
NS = 16
NW = NC * NS
BPW = B // NW
CHUNK = 128
NCHUNK = BPW // CHUNK
IDX_ROWS = 8

VC = 4096
NVB = 13
SPLIT = NVB * VC
LASTB = (VOCAB - 1) // VC


def _tp_body(lo_ref, hi_ref, o_ref):
    o_ref[...] = jnp.concatenate([lo_ref[...].T, hi_ref[...].T], axis=-1)


def _transpose_pack(tab_t):
    return pl.pallas_call(
        _tp_body,
        grid=(NVB,),
        in_specs=[
            pl.BlockSpec((EMB, VC), lambda i: (0, i)),
            pl.BlockSpec((EMB, VC),
                         lambda i: (0, jnp.minimum(i + NVB, LASTB))),
        ],
        out_specs=pl.BlockSpec((VC, 2 * EMB), lambda i: (i, 0)),
        out_shape=jax.ShapeDtypeStruct((SPLIT, 2 * EMB), jnp.float32),
    )(tab_t, tab_t)


@functools.lru_cache(maxsize=None)
def _make_sc_gather():
    mesh = plsc.VectorSubcoreMesh(core_axis_name="c", subcore_axis_name="s")

    @functools.partial(
        pl.kernel,
        mesh=mesh,
        out_type=jax.ShapeDtypeStruct((B, 2 * EMB), jnp.float32),
        scratch_types=[
            pltpu.VMEM((IDX_ROWS, CHUNK), jnp.int32),
            pltpu.VMEM((NCHUNK, CHUNK, 2 * EMB), jnp.float32),
            pltpu.SemaphoreType.DMA((NCHUNK,)),
            pltpu.SemaphoreType.DMA((NCHUNK,)),
        ],
    )
    def sc_gather(idx_hbm, tab_hbm, out_hbm, idx_v, rows_v, gsem, wsem):
        wid = lax.axis_index("s") * NC + lax.axis_index("c")
        base = wid * BPW
        pltpu.sync_copy(idx_hbm.at[wid], idx_v)
        gh = [None] * NCHUNK
        wh = [None] * NCHUNK
        for c in range(NCHUNK):
            gh[c] = pltpu.async_copy(
                tab_hbm.at[idx_v.at[c]], rows_v.at[c], gsem.at[c])
        for c in range(NCHUNK):
            gh[c].wait()
            wh[c] = pltpu.async_copy(
                rows_v.at[c], out_hbm.at[pl.ds(base + c * CHUNK, CHUNK)],
                wsem.at[c])
        for c in range(NCHUNK):
            wh[c].wait()

    return sc_gather


def _mm_body(g0, g1, g2, g3, g4, bsc_ref, w_ref, b_ref, o_ref):
    bm = o_ref.shape[0]
    half = (lax.broadcasted_iota(jnp.int32, (bm, 2 * EMB), 1) >= EMB)
    half = half.astype(jnp.int32)
    acc = b_ref[...].astype(jnp.float32)
    for t, g_ref in enumerate((g0, g1, g2, g3, g4)):
        par = (bsc_ref[:, t:t + 1] >= SPLIT).astype(jnp.int32)
        mask = half == par
        gt = jnp.where(mask, g_ref[...], 0.0)
        acc = acc + jnp.dot(gt, w_ref[t], preferred_element_type=jnp.float32)
    o_ref[...] = acc


def _project(gs, bsc, w_p, b_r):
    BM = 2048
    return pl.pallas_call(
        _mm_body,
        grid=(B // BM,),
        in_specs=[pl.BlockSpec((BM, 2 * EMB), lambda i: (i, 0))] * NT + [
            pl.BlockSpec((BM, NT), lambda i: (i, 0)),
            pl.BlockSpec((NT, 2 * EMB, HID), lambda i: (0, 0, 0)),
            pl.BlockSpec((1, HID), lambda i: (0, 0)),
        ],
        out_specs=pl.BlockSpec((BM, HID), lambda i: (i, 0)),
        out_shape=jax.ShapeDtypeStruct((B, HID), jnp.float32),
    )(*gs, bsc, w_p, b_r)


def kernel(batch_seq_cat, pop_tab, leng_tab, area_tab, lon_tab, lat_tab, W, b):
    bsc = batch_seq_cat.astype(jnp.int32)
    ridx = jnp.where(bsc < SPLIT, bsc, bsc - SPLIT)
    pidx = ridx.T.reshape(NT, NW, NCHUNK, CHUNK)
    pidx = jnp.concatenate(
        [pidx, jnp.zeros((NT, NW, IDX_ROWS - NCHUNK, CHUNK), jnp.int32)],
        axis=2)
    gather = _make_sc_gather()
    gs = []
    for t, tab in enumerate((pop_tab, leng_tab, area_tab, lon_tab, lat_tab)):
        gs.append(gather(pidx[t], _transpose_pack(tab.T)))
    w_p = jnp.concatenate([W.reshape(NT, EMB, HID)] * 2, axis=1)
    return _project(gs, bsc, w_p, b.reshape(1, HID))


# speedup vs baseline: 1.3020x; 1.0340x over previous
import functools

import jax
import jax.numpy as jnp
from jax import lax
from jax.experimental import pallas as pl
from jax.experimental.pallas import tpu as pltpu
from jax.experimental.pallas import tpu_sc as plsc

B = 16384
VOCAB = 100000
EMB = 64
HID = 64
NT = 5

NC = 2
NS = 16
NW = NC * NS
BPW = B // NW
CHUNK = 128
NCHUNK = BPW // CHUNK
IDX_ROWS = 8

VC = 6144
NVB = 9
SPLIT = NVB * VC
LASTB = (VOCAB - 1) // VC


def _tp_body(lo_ref, hi_ref, o_ref):
    o_ref[...] = jnp.concatenate([lo_ref[...].T, hi_ref[...].T], axis=-1)


def _transpose_pack(tab_t):
    return pl.pallas_call(
        _tp_body,
        grid=(NVB,),
        in_specs=[
            pl.BlockSpec((EMB, VC), lambda i: (0, i)),
            pl.BlockSpec((EMB, VC),
                         lambda i: (0, jnp.minimum(i + NVB, LASTB))),
        ],
        out_specs=pl.BlockSpec((VC, 2 * EMB), lambda i: (i, 0)),
        out_shape=jax.ShapeDtypeStruct((SPLIT, 2 * EMB), jnp.float32),
    )(tab_t, tab_t)


@functools.lru_cache(maxsize=None)
def _make_sc_gather():
    mesh = plsc.VectorSubcoreMesh(core_axis_name="c", subcore_axis_name="s")

    @functools.partial(
        pl.kernel,
        mesh=mesh,
        out_type=jax.ShapeDtypeStruct((B, 2 * EMB), jnp.float32),
        scratch_types=[
            pltpu.VMEM((IDX_ROWS, CHUNK), jnp.int32),
            pltpu.VMEM((NCHUNK, CHUNK, 2 * EMB), jnp.float32),
            pltpu.SemaphoreType.DMA((NCHUNK,)),
            pltpu.SemaphoreType.DMA((NCHUNK,)),
        ],
    )
    def sc_gather(idx_hbm, tab_hbm, out_hbm, idx_v, rows_v, gsem, wsem):
        wid = lax.axis_index("s") * NC + lax.axis_index("c")
        base = wid * BPW
        pltpu.sync_copy(idx_hbm.at[wid], idx_v)
        gh = [None] * NCHUNK
        wh = [None] * NCHUNK
        for c in range(NCHUNK):
            gh[c] = pltpu.async_copy(
                tab_hbm.at[idx_v.at[c]], rows_v.at[c], gsem.at[c])
        for c in range(NCHUNK):
            gh[c].wait()
            wh[c] = pltpu.async_copy(
                rows_v.at[c], out_hbm.at[pl.ds(base + c * CHUNK, CHUNK)],
                wsem.at[c])
        for c in range(NCHUNK):
            wh[c].wait()

    return sc_gather


def _mm_body(g0, g1, g2, g3, g4, bsc_ref, w_ref, b_ref, o_ref):
    bm = o_ref.shape[0]
    half = (lax.broadcasted_iota(jnp.int32, (bm, 2 * EMB), 1) >= EMB)
    half = half.astype(jnp.int32)
    acc = b_ref[...].astype(jnp.float32)
    for t, g_ref in enumerate((g0, g1, g2, g3, g4)):
        par = (bsc_ref[:, t:t + 1] >= SPLIT).astype(jnp.int32)
        mask = half == par
        gt = jnp.where(mask, g_ref[...], 0.0)
        acc = acc + jnp.dot(gt, w_ref[t], preferred_element_type=jnp.float32)
    o_ref[...] = acc


def _project(gs, bsc, w_p, b_r):
    BM = 2048
    return pl.pallas_call(
        _mm_body,
        grid=(B // BM,),
        in_specs=[pl.BlockSpec((BM, 2 * EMB), lambda i: (i, 0))] * NT + [
            pl.BlockSpec((BM, NT), lambda i: (i, 0)),
            pl.BlockSpec((NT, 2 * EMB, HID), lambda i: (0, 0, 0)),
            pl.BlockSpec((1, HID), lambda i: (0, 0)),
        ],
        out_specs=pl.BlockSpec((BM, HID), lambda i: (i, 0)),
        out_shape=jax.ShapeDtypeStruct((B, HID), jnp.float32),
    )(*gs, bsc, w_p, b_r)


def kernel(batch_seq_cat, pop_tab, leng_tab, area_tab, lon_tab, lat_tab, W, b):
    bsc = batch_seq_cat.astype(jnp.int32)
    ridx = jnp.where(bsc < SPLIT, bsc, bsc - SPLIT)
    pidx = ridx.T.reshape(NT, NW, NCHUNK, CHUNK)
    pidx = jnp.concatenate(
        [pidx, jnp.zeros((NT, NW, IDX_ROWS - NCHUNK, CHUNK), jnp.int32)],
        axis=2)
    gather = _make_sc_gather()
    gs = []
    for t, tab in enumerate((pop_tab, leng_tab, area_tab, lon_tab, lat_tab)):
        gs.append(gather(pidx[t], _transpose_pack(tab.T)))
    w_p = jnp.concatenate([W.reshape(NT, EMB, HID)] * 2, axis=1)
    return _project(gs, bsc, w_p, b.reshape(1, HID))
